# baseline (speedup 1.0000x reference)
import functools

import jax
import jax.numpy as jnp
from jax import lax
from jax.experimental import pallas as pl
from jax.experimental.pallas import tpu as pltpu
from jax.experimental.pallas import tpu_sc as plsc

NC = 2
NS = 16
NW = NC * NS

D = 2048
K = 8


@functools.partial(jax.jit, static_argnums=(2, 3))
def _emb_lookup(idx, table, b_per_w, nchunk):
    mesh = plsc.VectorSubcoreMesh(
        core_axis_name="c", subcore_axis_name="s",
        num_cores=NC, num_subcores=NS,
    )
    B = NW * b_per_w
    npairs = nchunk // 2

    @functools.partial(
        pl.kernel,
        out_type=jax.ShapeDtypeStruct((B, D), jnp.float32),
        mesh=mesh,
        scratch_types=[
            pltpu.VMEM((b_per_w,), jnp.int32),
            [pltpu.VMEM((K, D), jnp.float32)] * 2,
            [pltpu.VMEM((K, D), jnp.float32)] * 2,
            pltpu.VMEM_SHARED((NS, 2, K, D), jnp.float32),
            [pltpu.SemaphoreType.DMA] * 2,
            [pltpu.SemaphoreType.DMA] * 2,
            pltpu.SemaphoreType.DMA,
            [pltpu.SemaphoreType.DMA] * 2,
            [pltpu.SemaphoreType.DMA] * 2,
        ],
    )
    def body(idx_hbm, table_hbm, out_hbm, idx_v, bs, bd, sbuf,
             sis, sid_, smid, sos, sod):
        wid = lax.axis_index("s") * NC + lax.axis_index("c")
        sid = lax.axis_index("s")
        base = wid * b_per_w
        pltpu.sync_copy(idx_hbm.at[pl.ds(base, b_per_w)], idx_v)

        def gather(c, buf, sem):
            pltpu.async_copy(
                table_hbm.at[idx_v.at[pl.ds(c * K, K)]], buf, sem)

        def gather_wait(buf, sem):
            pltpu.make_async_copy(
                table_hbm.at[idx_v.at[pl.ds(0, K)]], buf, sem).wait()

        def wb_direct(c, m):
            pltpu.async_copy(
                bd[m], out_hbm.at[pl.ds(base + c * K, K)], sod[m])

        def wb_direct_wait(m):
            pltpu.make_async_copy(
                bd[m], out_hbm.at[pl.ds(base, K)], sod[m]).wait()

        def out_spmem(c, m):
            pltpu.async_copy(
                sbuf.at[sid, m], out_hbm.at[pl.ds(base + c * K, K)], sos[m])

        def out_spmem_wait(m):
            pltpu.make_async_copy(
                sbuf.at[sid, m], out_hbm.at[pl.ds(base, K)], sos[m]).wait()

        def pair(p, m, wait_os, wait_od, issue_d, issue_s):
            gather_wait(bs[m], sis[m])
            if wait_os:
                out_spmem_wait(m)
            pltpu.async_copy(bs[m], sbuf.at[sid, m], smid)
            gather_wait(bd[m], sid_[m])
            wb_direct(p * 2 + 1, m)
            if wait_od:
                wb_direct_wait(1 - m)
            if issue_d:
                gather(p * 2 + 3, bd[1 - m], sid_[1 - m])
            pltpu.make_async_copy(bs[m], sbuf.at[sid, m], smid).wait()
            out_spmem(p * 2, m)
            if issue_s:
                gather(p * 2 + 4, bs[m], sis[m])

        gather(0, bs[0], sis[0])
        gather(2, bs[1], sis[1])
        gather(1, bd[0], sid_[0])
        pair(0, 0, False, False, True, True)
        pair(1, 1, False, True, True, True)

        nrings = npairs // 2

        @pl.loop(1, nrings - 1)
        def ring(r):
            pair(2 * r, 0, True, True, True, True)
            pair(2 * r + 1, 1, True, True, True, True)

        pair(npairs - 2, 0, True, True, True, False)
        pair(npairs - 1, 1, True, True, False, False)
        out_spmem_wait(0)
        out_spmem_wait(1)
        wb_direct_wait(1)

    return body(idx, table)


def kernel(x, W):
    B = x.size
    b_per_w = B // NW
    nchunk = b_per_w // K
    out = _emb_lookup(x.reshape(-1), W, b_per_w, nchunk)
    return out.reshape(x.shape + (W.shape[1],))


# speedup vs baseline: 1.0101x; 1.0101x over previous
import functools

import jax
import jax.numpy as jnp
from jax import lax
from jax.experimental import pallas as pl
from jax.experimental.pallas import tpu as pltpu
from jax.experimental.pallas import tpu_sc as plsc

NC = 2
NS = 16
NW = NC * NS

D = 2048
K = 8
NB = 3


@functools.partial(jax.jit, static_argnums=(2, 3))
def _emb_lookup(idx, table, b_per_w, nchunk):
    mesh = plsc.VectorSubcoreMesh(
        core_axis_name="c", subcore_axis_name="s",
        num_cores=NC, num_subcores=NS,
    )
    B = NW * b_per_w

    @functools.partial(
        pl.kernel,
        out_type=jax.ShapeDtypeStruct((B, D), jnp.float32),
        mesh=mesh,
        scratch_types=[
            pltpu.VMEM((b_per_w,), jnp.int32),
            [pltpu.VMEM((K, D), jnp.float32)] * NB,
            pltpu.VMEM_SHARED((NS, NB, K, D), jnp.float32),
            [pltpu.SemaphoreType.DMA] * NB,
            [pltpu.SemaphoreType.DMA] * NB,
            [pltpu.SemaphoreType.DMA] * NB,
        ],
    )
    def body(idx_hbm, table_hbm, out_hbm, idx_v, bufs, sbuf, sin, smid, sout):
        wid = lax.axis_index("s") * NC + lax.axis_index("c")
        sid = lax.axis_index("s")
        base = wid * b_per_w
        pltpu.sync_copy(idx_hbm.at[pl.ds(base, b_per_w)], idx_v)

        def gather(c, b):
            pltpu.async_copy(
                table_hbm.at[idx_v.at[pl.ds(c * K, K)]], bufs[b], sin[b])

        def gather_wait(b):
            pltpu.make_async_copy(
                table_hbm.at[idx_v.at[pl.ds(0, K)]], bufs[b], sin[b]).wait()

        def mid(b):
            pltpu.async_copy(bufs[b], sbuf.at[sid, b], smid[b])

        def mid_wait(b):
            pltpu.make_async_copy(bufs[b], sbuf.at[sid, b], smid[b]).wait()

        def out(c, b):
            pltpu.async_copy(
                sbuf.at[sid, b], out_hbm.at[pl.ds(base + c * K, K)], sout[b])

        def out_wait(b):
            pltpu.make_async_copy(
                sbuf.at[sid, b], out_hbm.at[pl.ds(base, K)], sout[b]).wait()

        def step(c, b, wait_out, do_prev, issue_in):
            gather_wait(b)
            if wait_out:
                out_wait(b)
            mid(b)
            if do_prev:
                pb = (b + NB - 1) % NB
                mid_wait(pb)
                out(c - 1, pb)
                if issue_in:
                    gather(c + 2, pb)

        gather(0, 0)
        gather(1, 1)
        step(0, 0, False, False, False)
        gather(2, 2)
        step(1, 1, False, True, True)
        step(2, 2, False, True, True)

        nfull = (nchunk - 4 - 3) // NB + 1

        @pl.loop(1, nfull)
        def ring(r):
            for b in range(NB):
                step(NB * r + b, b, True, True, True)

        for c in range(NB * nfull, nchunk):
            step(c, c % NB, True, True, c + 2 < nchunk)
        last = (nchunk - 1) % NB
        mid_wait(last)
        out(nchunk - 1, last)
        out_wait((nchunk - 3) % NB)
        out_wait((nchunk - 2) % NB)
        out_wait(last)

    return body(idx, table)


def kernel(x, W):
    B = x.size
    b_per_w = B // NW
    nchunk = b_per_w // K
    out = _emb_lookup(x.reshape(-1), W, b_per_w, nchunk)
    return out.reshape(x.shape + (W.shape[1],))
